# Initial kernel scaffold; baseline (speedup 1.0000x reference)
import jax
import jax.numpy as jnp
from jax.experimental import pallas as pl


def kernel(x, edge_index, W1, b1, W2, b2):
    raise NotImplementedError("write your pallas kernel here")


# speedup vs baseline: 3.6921x; 3.6921x over previous
import functools

import jax
import jax.numpy as jnp
from jax import lax
from jax.experimental import pallas as pl
from jax.experimental.pallas import tpu as pltpu
from jax.experimental.pallas import tpu_sc as plsc

NN = 10000
NPAD = 10240
D = 256
DH = 128
E = 160000
EPAD = 163840
EROWS = EPAD // 128
DEGW = 16

ROWS_DEG = EROWS // 32
ROWS_AGG = EROWS // 16
SLAB = NPAD // 16

_MESH = plsc.VectorSubcoreMesh(core_axis_name="c", subcore_axis_name="s")


@functools.partial(
    pl.kernel,
    out_type=jax.ShapeDtypeStruct((2 * NPAD, 128), jnp.float32),
    mesh=_MESH,
    scratch_types=[
        pltpu.VMEM((ROWS_AGG, 128), jnp.int32),
        pltpu.VMEM((128, 128), jnp.float32),
        pltpu.VMEM_SHARED((NPAD, 128), jnp.float32),
    ],
)
def _deg(idx2_hbm, out_hbm, idx_v, ones_v, deg_sh):
    c = lax.axis_index("c")
    s = lax.axis_index("s")

    @pl.loop(0, 128)
    def _(i):
        @pl.loop(0, 128, step=16)
        def _(k):
            ones_v.at[i, pl.ds(k, 16)][...] = jnp.zeros((16,), jnp.float32)

    @pl.loop(0, SLAB // 128)
    def _(k):
        pltpu.sync_copy(ones_v, deg_sh.at[pl.ds(s * SLAB + k * 128, 128)])

    @pl.loop(0, 128)
    def _(i):
        @pl.loop(0, 128, step=16)
        def _(k):
            ones_v.at[i, pl.ds(k, 16)][...] = jnp.ones((16,), jnp.float32)

    pltpu.sync_copy(idx2_hbm.at[pl.ds(c * EROWS + s * ROWS_AGG, ROWS_AGG)],
                    idx_v)
    plsc.subcore_barrier()

    @pl.loop(0, ROWS_AGG)
    def _(j):
        pltpu.sync_copy(ones_v, deg_sh.at[idx_v.at[j]], add=True)

    plsc.subcore_barrier()
    pltpu.sync_copy(deg_sh.at[pl.ds(s * SLAB, SLAB)],
                    out_hbm.at[pl.ds(c * NPAD + s * SLAB, SLAB)])


@functools.partial(
    pl.kernel,
    out_type=jax.ShapeDtypeStruct((2 * NPAD, DH), jnp.float32),
    mesh=_MESH,
    scratch_types=[
        pltpu.VMEM((ROWS_AGG, 128), jnp.int32),
        pltpu.VMEM((ROWS_AGG, 128), jnp.int32),
        pltpu.VMEM((128, DH), jnp.float32),
        pltpu.VMEM_SHARED((NPAD, DH), jnp.float32),
    ],
)
def _agg(init_hbm, h_hbm, snd2_hbm, rcv_hbm, out_hbm,
         snd_v, rcv_v, rows_v, acc_sh):
    c = lax.axis_index("c")
    s = lax.axis_index("s")

    pltpu.sync_copy(init_hbm.at[pl.ds(c * NPAD + s * SLAB, SLAB)],
                    acc_sh.at[pl.ds(s * SLAB, SLAB)])

    pltpu.sync_copy(snd2_hbm.at[pl.ds(c * EROWS + s * ROWS_AGG, ROWS_AGG)],
                    snd_v)
    pltpu.sync_copy(rcv_hbm.at[pl.ds(s * ROWS_AGG, ROWS_AGG)], rcv_v)
    plsc.subcore_barrier()

    @pl.loop(0, ROWS_AGG)
    def _(j):
        pltpu.sync_copy(h_hbm.at[snd_v.at[j]], rows_v)
        pltpu.sync_copy(rows_v, acc_sh.at[rcv_v.at[j]], add=True)

    plsc.subcore_barrier()
    pltpu.sync_copy(acc_sh.at[pl.ds(s * SLAB, SLAB)],
                    out_hbm.at[pl.ds(c * NPAD + s * SLAB, SLAB)])


def _mm1(x_pad, W1, b1, degs):
    def body(x_ref, w_ref, b_ref, d_ref, o_ref):
        ds1 = d_ref[0, :] + 1.0
        scale = lax.rsqrt(ds1)
        h = jnp.dot(x_ref[...], w_ref[...], preferred_element_type=jnp.float32)
        h = (h + b_ref[...]) * scale[:, None]
        o_ref[0] = h[:, :DH]
        o_ref[1] = h[:, DH:]

    return pl.pallas_call(
        body,
        grid=(NPAD // 256,),
        in_specs=[
            pl.BlockSpec((256, D), lambda i: (i, 0)),
            pl.BlockSpec((D, D), lambda i: (0, 0)),
            pl.BlockSpec((1, D), lambda i: (0, 0)),
            pl.BlockSpec((2, 256), lambda i: (0, i)),
        ],
        out_specs=pl.BlockSpec((2, 256, DH), lambda i: (0, i, 0)),
        out_shape=jax.ShapeDtypeStruct((2, NPAD, DH), jnp.float32),
    )(x_pad, W1, b1, degs)


def _mid(acc1, degs, W2, b2):
    def body(a_ref, d_ref, w_ref, b_ref, o_ref):
        dr1 = d_ref[1, :] + 1.0
        ds2 = jnp.maximum(d_ref[0, :], 1.0)
        pre = jnp.concatenate([a_ref[0], a_ref[1]], axis=1)
        pre = pre * lax.rsqrt(dr1)[:, None]
        u = jnp.dot(pre, w_ref[...], preferred_element_type=jnp.float32)
        u = (u + b_ref[...]) * lax.rsqrt(ds2)[:, None]
        o_ref[0] = u[:, :DH]
        o_ref[1] = u[:, DH:]

    return pl.pallas_call(
        body,
        grid=(NPAD // 256,),
        in_specs=[
            pl.BlockSpec((2, 256, DH), lambda i: (0, i, 0)),
            pl.BlockSpec((2, 256), lambda i: (0, i)),
            pl.BlockSpec((D, D), lambda i: (0, 0)),
            pl.BlockSpec((1, D), lambda i: (0, 0)),
        ],
        out_specs=pl.BlockSpec((2, 256, DH), lambda i: (0, i, 0)),
        out_shape=jax.ShapeDtypeStruct((2, NPAD, DH), jnp.float32),
    )(acc1, degs, W2, b2)


def _fin(acc2, degs):
    def body(a_ref, d_ref, o_ref):
        dr2 = jnp.maximum(d_ref[1, :], 1.0)
        o_ref[...] = (jnp.concatenate([a_ref[0], a_ref[1]], axis=1)
                      * lax.rsqrt(dr2)[:, None])

    return pl.pallas_call(
        body,
        grid=(NPAD // 256,),
        in_specs=[
            pl.BlockSpec((2, 256, DH), lambda i: (0, i, 0)),
            pl.BlockSpec((2, 256), lambda i: (0, i)),
        ],
        out_specs=pl.BlockSpec((256, D), lambda i: (i, 0)),
        out_shape=jax.ShapeDtypeStruct((NPAD, D), jnp.float32),
    )(acc2, degs)


def kernel(x, edge_index, W1, b1, W2, b2):
    snd = edge_index[0].astype(jnp.int32)
    rcv = edge_index[1].astype(jnp.int32)
    pad = jnp.full((EPAD - E,), NN, jnp.int32)
    snd = jnp.concatenate([snd, pad]).reshape(EROWS, 128)
    rcv = jnp.concatenate([rcv, pad]).reshape(EROWS, 128)
    snd2 = jnp.concatenate([snd, snd + NPAD], axis=0)
    x_pad = jnp.pad(x, ((0, NPAD - NN), (0, 0)))
    b1r = b1.reshape(1, D)
    b2r = b2.reshape(1, D)

    idx2 = jnp.concatenate([snd, rcv], axis=0)
    degw = _deg(idx2)
    degs = degw.reshape(2, NPAD, 128)[..., 0]

    h1 = _mm1(x_pad, W1, b1r, degs)
    h1f = h1.reshape(2 * NPAD, DH)
    acc1 = _agg(h1f, h1f, snd2, rcv)
    h2 = _mid(acc1.reshape(2, NPAD, DH), degs, W2, b2r)
    zeros = jnp.zeros((2 * NPAD, DH), jnp.float32)
    acc2 = _agg(zeros, h2.reshape(2 * NPAD, DH), snd2, rcv)
    out = _fin(acc2.reshape(2, NPAD, DH), degs)
    return out[:NN]


# speedup vs baseline: 4.0946x; 1.1090x over previous
import functools

import jax
import jax.numpy as jnp
from jax import lax
from jax.experimental import pallas as pl
from jax.experimental.pallas import tpu as pltpu
from jax.experimental.pallas import tpu_sc as plsc

NN = 10000
NPAD = 10240
D = 256
DH = 128
E = 160000
EPAD = 163840
EROWS = EPAD // 128
DEGW = 16

ROWS_DEG = EROWS // 32
ROWS_AGG = EROWS // 16
SLAB = NPAD // 16

_MESH = plsc.VectorSubcoreMesh(core_axis_name="c", subcore_axis_name="s")


@functools.partial(
    pl.kernel,
    out_type=jax.ShapeDtypeStruct((2 * NPAD, 128), jnp.float32),
    mesh=_MESH,
    scratch_types=[
        pltpu.VMEM((ROWS_AGG, 128), jnp.int32),
        pltpu.VMEM((128, 128), jnp.float32),
        pltpu.VMEM_SHARED((NPAD, 128), jnp.float32),
    ],
)
def _deg(idx2_hbm, out_hbm, idx_v, ones_v, deg_sh):
    c = lax.axis_index("c")
    s = lax.axis_index("s")

    @pl.loop(0, 128)
    def _(i):
        @pl.loop(0, 128, step=16)
        def _(k):
            ones_v.at[i, pl.ds(k, 16)][...] = jnp.zeros((16,), jnp.float32)

    @pl.loop(0, SLAB // 128)
    def _(k):
        pltpu.sync_copy(ones_v, deg_sh.at[pl.ds(s * SLAB + k * 128, 128)])

    @pl.loop(0, 128)
    def _(i):
        @pl.loop(0, 128, step=16)
        def _(k):
            ones_v.at[i, pl.ds(k, 16)][...] = jnp.ones((16,), jnp.float32)

    pltpu.sync_copy(idx2_hbm.at[pl.ds(c * EROWS + s * ROWS_AGG, ROWS_AGG)],
                    idx_v)
    plsc.subcore_barrier()

    @pl.loop(0, ROWS_AGG)
    def _(j):
        pltpu.sync_copy(ones_v, deg_sh.at[idx_v.at[j]], add=True)

    plsc.subcore_barrier()
    pltpu.sync_copy(deg_sh.at[pl.ds(s * SLAB, SLAB)],
                    out_hbm.at[pl.ds(c * NPAD + s * SLAB, SLAB)])


_NBUF = 2
_G = 8
_NG = ROWS_AGG // _G


@functools.partial(
    pl.kernel,
    out_type=jax.ShapeDtypeStruct((2 * NPAD, DH), jnp.float32),
    mesh=_MESH,
    scratch_types=(
        [pltpu.VMEM((_G, 128), jnp.int32),
         pltpu.VMEM((_G, 128), jnp.int32),
         pltpu.VMEM_SHARED((NPAD, DH), jnp.float32)]
        + [pltpu.VMEM((128, DH), jnp.float32)] * _NBUF
        + [pltpu.SemaphoreType.DMA] * (2 * _NBUF)
    ),
)
def _agg(init_hbm, h_hbm, snd2_hbm, rcv_hbm, out_hbm,
         snd_g, rcv_g, acc_sh, *bufs_sems):
    bufs = bufs_sems[:_NBUF]
    gsems = bufs_sems[_NBUF:2 * _NBUF]
    ssems = bufs_sems[2 * _NBUF:]
    c = lax.axis_index("c")
    s = lax.axis_index("s")

    pltpu.sync_copy(init_hbm.at[pl.ds(c * NPAD + s * SLAB, SLAB)],
                    acc_sh.at[pl.ds(s * SLAB, SLAB)])
    plsc.subcore_barrier()

    base2 = c * EROWS + s * ROWS_AGG
    base1 = s * ROWS_AGG

    @pl.loop(0, _NG)
    def _(g):
        pltpu.sync_copy(snd2_hbm.at[pl.ds(base2 + g * _G, _G)], snd_g)
        pltpu.sync_copy(rcv_hbm.at[pl.ds(base1 + g * _G, _G)], rcv_g)

        for b in range(_NBUF):
            pltpu.async_copy(h_hbm.at[snd_g.at[b]], bufs[b], gsems[b])

        @pl.loop(0, _G, step=_NBUF)
        def _(k):
            for b in range(_NBUF):
                kk = k + b
                pltpu.make_async_copy(h_hbm.at[snd_g.at[kk]], bufs[b],
                                      gsems[b]).wait()
                pltpu.async_copy(bufs[b], acc_sh.at[rcv_g.at[kk]], ssems[b],
                                 add=True)

                @pl.when(kk + _NBUF < _G)
                def _():
                    pltpu.make_async_copy(bufs[b], acc_sh.at[rcv_g.at[kk]],
                                          ssems[b]).wait()
                    pltpu.async_copy(h_hbm.at[snd_g.at[kk + _NBUF]], bufs[b],
                                     gsems[b])

        for b in range(_NBUF):
            pltpu.make_async_copy(bufs[b], acc_sh.at[rcv_g.at[_G - _NBUF + b]],
                                  ssems[b]).wait()

    plsc.subcore_barrier()
    pltpu.sync_copy(acc_sh.at[pl.ds(s * SLAB, SLAB)],
                    out_hbm.at[pl.ds(c * NPAD + s * SLAB, SLAB)])


def _mm1(x_pad, W1, b1, degs):
    def body(x_ref, w_ref, b_ref, d_ref, o_ref):
        ds1 = d_ref[0, :] + 1.0
        scale = lax.rsqrt(ds1)
        h = jnp.dot(x_ref[...], w_ref[...], preferred_element_type=jnp.float32)
        h = (h + b_ref[...]) * scale[:, None]
        o_ref[0] = h[:, :DH]
        o_ref[1] = h[:, DH:]

    return pl.pallas_call(
        body,
        grid=(NPAD // 256,),
        in_specs=[
            pl.BlockSpec((256, D), lambda i: (i, 0)),
            pl.BlockSpec((D, D), lambda i: (0, 0)),
            pl.BlockSpec((1, D), lambda i: (0, 0)),
            pl.BlockSpec((2, 256), lambda i: (0, i)),
        ],
        out_specs=pl.BlockSpec((2, 256, DH), lambda i: (0, i, 0)),
        out_shape=jax.ShapeDtypeStruct((2, NPAD, DH), jnp.float32),
    )(x_pad, W1, b1, degs)


def _mid(acc1, degs, W2, b2):
    def body(a_ref, d_ref, w_ref, b_ref, o_ref):
        dr1 = d_ref[1, :] + 1.0
        ds2 = jnp.maximum(d_ref[0, :], 1.0)
        pre = jnp.concatenate([a_ref[0], a_ref[1]], axis=1)
        pre = pre * lax.rsqrt(dr1)[:, None]
        u = jnp.dot(pre, w_ref[...], preferred_element_type=jnp.float32)
        u = (u + b_ref[...]) * lax.rsqrt(ds2)[:, None]
        o_ref[0] = u[:, :DH]
        o_ref[1] = u[:, DH:]

    return pl.pallas_call(
        body,
        grid=(NPAD // 256,),
        in_specs=[
            pl.BlockSpec((2, 256, DH), lambda i: (0, i, 0)),
            pl.BlockSpec((2, 256), lambda i: (0, i)),
            pl.BlockSpec((D, D), lambda i: (0, 0)),
            pl.BlockSpec((1, D), lambda i: (0, 0)),
        ],
        out_specs=pl.BlockSpec((2, 256, DH), lambda i: (0, i, 0)),
        out_shape=jax.ShapeDtypeStruct((2, NPAD, DH), jnp.float32),
    )(acc1, degs, W2, b2)


def _fin(acc2, degs):
    def body(a_ref, d_ref, o_ref):
        dr2 = jnp.maximum(d_ref[1, :], 1.0)
        o_ref[...] = (jnp.concatenate([a_ref[0], a_ref[1]], axis=1)
                      * lax.rsqrt(dr2)[:, None])

    return pl.pallas_call(
        body,
        grid=(NPAD // 256,),
        in_specs=[
            pl.BlockSpec((2, 256, DH), lambda i: (0, i, 0)),
            pl.BlockSpec((2, 256), lambda i: (0, i)),
        ],
        out_specs=pl.BlockSpec((256, D), lambda i: (i, 0)),
        out_shape=jax.ShapeDtypeStruct((NPAD, D), jnp.float32),
    )(acc2, degs)


def kernel(x, edge_index, W1, b1, W2, b2):
    snd = edge_index[0].astype(jnp.int32)
    rcv = edge_index[1].astype(jnp.int32)
    pad = jnp.full((EPAD - E,), NN, jnp.int32)
    snd = jnp.concatenate([snd, pad]).reshape(EROWS, 128)
    rcv = jnp.concatenate([rcv, pad]).reshape(EROWS, 128)
    snd2 = jnp.concatenate([snd, snd + NPAD], axis=0)
    x_pad = jnp.pad(x, ((0, NPAD - NN), (0, 0)))
    b1r = b1.reshape(1, D)
    b2r = b2.reshape(1, D)

    idx2 = jnp.concatenate([snd, rcv], axis=0)
    degw = _deg(idx2)
    degs = degw.reshape(2, NPAD, 128)[..., 0]

    h1 = _mm1(x_pad, W1, b1r, degs)
    h1f = h1.reshape(2 * NPAD, DH)
    acc1 = _agg(h1f, h1f, snd2, rcv)
    h2 = _mid(acc1.reshape(2, NPAD, DH), degs, W2, b2r)
    zeros = jnp.zeros((2 * NPAD, DH), jnp.float32)
    acc2 = _agg(zeros, h2.reshape(2 * NPAD, DH), snd2, rcv)
    out = _fin(acc2.reshape(2, NPAD, DH), degs)
    return out[:NN]


# speedup vs baseline: 4.2067x; 1.0274x over previous
import functools

import jax
import jax.numpy as jnp
from jax import lax
from jax.experimental import pallas as pl
from jax.experimental.pallas import tpu as pltpu
from jax.experimental.pallas import tpu_sc as plsc

NN = 10000
NPAD = 10240
D = 256
DH = 128
E = 160000
EPAD = 163840
EROWS = EPAD // 128
DEGW = 16

ROWS_DEG = EROWS // 32
ROWS_AGG = EROWS // 16
SLAB = NPAD // 16

_MESH = plsc.VectorSubcoreMesh(core_axis_name="c", subcore_axis_name="s")


@functools.partial(
    pl.kernel,
    out_type=jax.ShapeDtypeStruct((2 * NPAD, 128), jnp.float32),
    mesh=_MESH,
    scratch_types=[
        pltpu.VMEM((ROWS_AGG, 128), jnp.int32),
        pltpu.VMEM((128, 128), jnp.float32),
        pltpu.VMEM_SHARED((NPAD, 128), jnp.float32),
    ],
)
def _deg(idx2_hbm, out_hbm, idx_v, ones_v, deg_sh):
    c = lax.axis_index("c")
    s = lax.axis_index("s")

    @pl.loop(0, 128)
    def _(i):
        @pl.loop(0, 128, step=16)
        def _(k):
            ones_v.at[i, pl.ds(k, 16)][...] = jnp.zeros((16,), jnp.float32)

    @pl.loop(0, SLAB // 128)
    def _(k):
        pltpu.sync_copy(ones_v, deg_sh.at[pl.ds(s * SLAB + k * 128, 128)])

    @pl.loop(0, 128)
    def _(i):
        @pl.loop(0, 128, step=16)
        def _(k):
            ones_v.at[i, pl.ds(k, 16)][...] = jnp.ones((16,), jnp.float32)

    pltpu.sync_copy(idx2_hbm.at[pl.ds(c * EROWS + s * ROWS_AGG, ROWS_AGG)],
                    idx_v)
    plsc.subcore_barrier()

    @pl.loop(0, ROWS_AGG)
    def _(j):
        pltpu.sync_copy(ones_v, deg_sh.at[idx_v.at[j]], add=True)

    plsc.subcore_barrier()
    pltpu.sync_copy(deg_sh.at[pl.ds(s * SLAB, SLAB)],
                    out_hbm.at[pl.ds(c * NPAD + s * SLAB, SLAB)])


_NBUF = 2
_G = 16
_NG = ROWS_AGG // _G


@functools.partial(
    pl.kernel,
    out_type=jax.ShapeDtypeStruct((2 * NPAD, DH), jnp.float32),
    mesh=_MESH,
    scratch_types=(
        [pltpu.VMEM((_G, 128), jnp.int32),
         pltpu.VMEM((_G, 128), jnp.int32),
         pltpu.VMEM_SHARED((NPAD, DH), jnp.float32)]
        + [pltpu.VMEM((128, DH), jnp.float32)] * _NBUF
        + [pltpu.SemaphoreType.DMA] * (2 * _NBUF)
    ),
)
def _agg(init_hbm, h_hbm, snd2_hbm, rcv_hbm, out_hbm,
         snd_g, rcv_g, acc_sh, *bufs_sems):
    bufs = bufs_sems[:_NBUF]
    gsems = bufs_sems[_NBUF:2 * _NBUF]
    ssems = bufs_sems[2 * _NBUF:]
    c = lax.axis_index("c")
    s = lax.axis_index("s")

    pltpu.sync_copy(init_hbm.at[pl.ds(c * NPAD + s * SLAB, SLAB)],
                    acc_sh.at[pl.ds(s * SLAB, SLAB)])
    plsc.subcore_barrier()

    base2 = c * EROWS + s * ROWS_AGG
    base1 = s * ROWS_AGG

    @pl.loop(0, _NG)
    def _(g):
        pltpu.sync_copy(snd2_hbm.at[pl.ds(base2 + g * _G, _G)], snd_g)
        pltpu.sync_copy(rcv_hbm.at[pl.ds(base1 + g * _G, _G)], rcv_g)

        for b in range(_NBUF):
            pltpu.async_copy(h_hbm.at[snd_g.at[b]], bufs[b], gsems[b])

        @pl.loop(0, _G, step=_NBUF)
        def _(k):
            for b in range(_NBUF):
                kk = k + b
                pltpu.make_async_copy(h_hbm.at[snd_g.at[kk]], bufs[b],
                                      gsems[b]).wait()
                pltpu.async_copy(bufs[b], acc_sh.at[rcv_g.at[kk]], ssems[b],
                                 add=True)

                @pl.when(kk + _NBUF < _G)
                def _():
                    pltpu.make_async_copy(bufs[b], acc_sh.at[rcv_g.at[kk]],
                                          ssems[b]).wait()
                    pltpu.async_copy(h_hbm.at[snd_g.at[kk + _NBUF]], bufs[b],
                                     gsems[b])

        for b in range(_NBUF):
            pltpu.make_async_copy(bufs[b], acc_sh.at[rcv_g.at[_G - _NBUF + b]],
                                  ssems[b]).wait()

    plsc.subcore_barrier()
    pltpu.sync_copy(acc_sh.at[pl.ds(s * SLAB, SLAB)],
                    out_hbm.at[pl.ds(c * NPAD + s * SLAB, SLAB)])


def _mm1(x_pad, W1, b1):
    def body(x_ref, w_ref, b_ref, o_ref):
        h = jnp.dot(x_ref[...], w_ref[...], preferred_element_type=jnp.float32)
        o_ref[...] = h + b_ref[...]

    return pl.pallas_call(
        body,
        grid=(NPAD // 256,),
        in_specs=[
            pl.BlockSpec((256, D), lambda i: (i, 0)),
            pl.BlockSpec((D, D), lambda i: (0, 0)),
            pl.BlockSpec((1, D), lambda i: (0, 0)),
        ],
        out_specs=pl.BlockSpec((256, D), lambda i: (i, 0)),
        out_shape=jax.ShapeDtypeStruct((NPAD, D), jnp.float32),
    )(x_pad, W1, b1)


def _scale1(h_raw, degs):
    def body(h_ref, d_ref, o_ref):
        scale = lax.rsqrt(d_ref[0, :] + 1.0)
        h = h_ref[...] * scale[:, None]
        o_ref[0] = h[:, :DH]
        o_ref[1] = h[:, DH:]

    return pl.pallas_call(
        body,
        grid=(NPAD // 256,),
        in_specs=[
            pl.BlockSpec((256, D), lambda i: (i, 0)),
            pl.BlockSpec((2, 256), lambda i: (0, i)),
        ],
        out_specs=pl.BlockSpec((2, 256, DH), lambda i: (0, i, 0)),
        out_shape=jax.ShapeDtypeStruct((2, NPAD, DH), jnp.float32),
    )(h_raw, degs)


def _mid(acc1, degs, W2, b2):
    def body(a_ref, d_ref, w_ref, b_ref, o_ref):
        dr1 = d_ref[1, :] + 1.0
        ds2 = jnp.maximum(d_ref[0, :], 1.0)
        pre = jnp.concatenate([a_ref[0], a_ref[1]], axis=1)
        pre = pre * lax.rsqrt(dr1)[:, None]
        u = jnp.dot(pre, w_ref[...], preferred_element_type=jnp.float32)
        u = (u + b_ref[...]) * lax.rsqrt(ds2)[:, None]
        o_ref[0] = u[:, :DH]
        o_ref[1] = u[:, DH:]

    return pl.pallas_call(
        body,
        grid=(NPAD // 256,),
        in_specs=[
            pl.BlockSpec((2, 256, DH), lambda i: (0, i, 0)),
            pl.BlockSpec((2, 256), lambda i: (0, i)),
            pl.BlockSpec((D, D), lambda i: (0, 0)),
            pl.BlockSpec((1, D), lambda i: (0, 0)),
        ],
        out_specs=pl.BlockSpec((2, 256, DH), lambda i: (0, i, 0)),
        out_shape=jax.ShapeDtypeStruct((2, NPAD, DH), jnp.float32),
    )(acc1, degs, W2, b2)


def _fin(acc2, degs):
    def body(a_ref, d_ref, o_ref):
        dr2 = jnp.maximum(d_ref[1, :], 1.0)
        o_ref[...] = (jnp.concatenate([a_ref[0], a_ref[1]], axis=1)
                      * lax.rsqrt(dr2)[:, None])

    return pl.pallas_call(
        body,
        grid=(NPAD // 256,),
        in_specs=[
            pl.BlockSpec((2, 256, DH), lambda i: (0, i, 0)),
            pl.BlockSpec((2, 256), lambda i: (0, i)),
        ],
        out_specs=pl.BlockSpec((256, D), lambda i: (i, 0)),
        out_shape=jax.ShapeDtypeStruct((NPAD, D), jnp.float32),
    )(acc2, degs)


def kernel(x, edge_index, W1, b1, W2, b2):
    snd = edge_index[0].astype(jnp.int32)
    rcv = edge_index[1].astype(jnp.int32)
    pad = jnp.full((EPAD - E,), NN, jnp.int32)
    snd = jnp.concatenate([snd, pad]).reshape(EROWS, 128)
    rcv = jnp.concatenate([rcv, pad]).reshape(EROWS, 128)
    snd2 = jnp.concatenate([snd, snd + NPAD], axis=0)
    x_pad = jnp.pad(x, ((0, NPAD - NN), (0, 0)))
    b1r = b1.reshape(1, D)
    b2r = b2.reshape(1, D)

    idx2 = jnp.concatenate([snd, rcv], axis=0)
    degw = _deg(idx2)
    degs = degw.reshape(2, NPAD, 128)[..., 0]

    h1_raw = _mm1(x_pad, W1, b1r)
    h1 = _scale1(h1_raw, degs)
    h1f = h1.reshape(2 * NPAD, DH)
    acc1 = _agg(h1f, h1f, snd2, rcv)
    h2 = _mid(acc1.reshape(2, NPAD, DH), degs, W2, b2r)
    zeros = jnp.zeros((2 * NPAD, DH), jnp.float32)
    acc2 = _agg(zeros, h2.reshape(2 * NPAD, DH), snd2, rcv)
    out = _fin(acc2.reshape(2, NPAD, DH), degs)
    return out[:NN]
